# baseline (speedup 1.0000x reference)
import functools

import numpy as np
import jax
import jax.numpy as jnp
from jax import lax
from jax.experimental import pallas as pl
from jax.experimental.pallas import tpu as pltpu
from jax.experimental.pallas import tpu_sc as plsc

N_NODES = 10000
N_EDGES = 320000
D = 128

NC = 2
NS = 16
N_TILES = NC * NS

EDGES_PER_TILE = N_EDGES // N_TILES
BATCH = 96
HALF = BATCH // 2
N_BATCHES = EDGES_PER_TILE // BATCH
TAIL = EDGES_PER_TILE - N_BATCHES * BATCH

_PERM = np.concatenate(
    [32 * blk + np.r_[np.arange(0, 32, 2), np.arange(1, 32, 2)]
     for blk in range(D // 32)])
N_PAD = 10240
ROWS_PER_TILE = N_PAD // NS
CHUNK = 40
N_CHUNKS = ROWS_PER_TILE // CHUNK


@functools.partial(
    pl.kernel,
    mesh=plsc.VectorSubcoreMesh(core_axis_name="c", subcore_axis_name="s"),
    compiler_params=pltpu.CompilerParams(use_tc_tiling_on_sc=False,
                                         needs_layout_passes=False),
    out_type=jax.ShapeDtypeStruct((NC, N_PAD, D), jnp.float32),
    scratch_types=[
        pltpu.VMEM((EDGES_PER_TILE,), jnp.int32),
        pltpu.VMEM((EDGES_PER_TILE,), jnp.int32),
        pltpu.VMEM((2, BATCH, D), jnp.bfloat16),
        pltpu.VMEM((2, HALF, D), jnp.float32),
        pltpu.VMEM_SHARED((N_PAD, D), jnp.float32),
        [pltpu.SemaphoreType.DMA] * 2,
        [pltpu.SemaphoreType.DMA] * 2,
    ],
)
def _aggregate(ei_hbm, feat_hbm, out_hbm,
               src_v, dst_v, rows_v, fbuf_v, acc_sh, gsems, ssems):
    c = lax.axis_index("c")
    s = lax.axis_index("s")
    w = c * NS + s

    zeros16 = jnp.zeros((16,), jnp.float32)

    def zero_body(i, _):
        r = i // (D // 16)
        col = (i % (D // 16)) * 16
        fbuf_v[0, r, pl.ds(col, 16)] = zeros16
        return 0

    lax.fori_loop(0, CHUNK * (D // 16), zero_body, 0)

    row0 = s * ROWS_PER_TILE
    zstage = fbuf_v.at[0].at[pl.ds(0, CHUNK)]
    for j in range(N_CHUNKS):
        pltpu.sync_copy(zstage, acc_sh.at[pl.ds(row0 + j * CHUNK, CHUNK)])

    ebase = w * EDGES_PER_TILE
    pltpu.sync_copy(ei_hbm.at[0, pl.ds(ebase, EDGES_PER_TILE)], src_v)
    pltpu.sync_copy(ei_hbm.at[1, pl.ds(ebase, EDGES_PER_TILE)], dst_v)
    plsc.subcore_barrier()

    def gather_start(i, slot):
        idx = src_v.at[pl.ds(i * BATCH, BATCH)]
        pltpu.async_copy(feat_hbm.at[idx], rows_v.at[slot], gsems[slot])

    def gather_wait(i, slot):
        idx = src_v.at[pl.ds(i * BATCH, BATCH)]
        pltpu.make_async_copy(feat_hbm.at[idx], rows_v.at[slot],
                              gsems[slot]).wait()

    def widen_half(slot, h):
        def conv_body(t, _):
            r = t // (D // 32)
            g = (t % (D // 32)) * 32
            v = rows_v[slot, h * HALF + r, pl.ds(g, 32)]
            a, b = plsc.unpack(v, format=plsc.PackFormat.INTERLEAVED)
            fbuf_v[h, r, pl.ds(g, 16)] = a
            fbuf_v[h, r, pl.ds(g + 16, 16)] = b
            return 0

        lax.fori_loop(0, HALF * (D // 32), conv_body, 0)

    def scatter_start(i, h):
        didx = dst_v.at[pl.ds(i * BATCH + h * HALF, HALF)]
        pltpu.async_copy(fbuf_v.at[h], acc_sh.at[didx], ssems[h], add=True)

    def scatter_wait(i, h):
        didx = dst_v.at[pl.ds(i * BATCH + h * HALF, HALF)]
        pltpu.make_async_copy(fbuf_v.at[h], acc_sh.at[didx],
                              ssems[h]).wait()

    gather_start(0, 0)
    gather_start(1, 1)

    def pair_body(k, _):
        for slot in range(2):
            i = 2 * k + slot
            gather_wait(i, slot)
            for h in range(2):
                @pl.when(i > 0)
                def _():
                    scatter_wait(i - 1, h)
                widen_half(slot, h)
                scatter_start(i, h)

            @pl.when(i + 2 < N_BATCHES)
            def _():
                gather_start(i + 2, slot)
        return 0

    lax.fori_loop(0, N_BATCHES // 2, pair_body, 0)
    tbase = N_BATCHES * BATCH
    tidx = src_v.at[pl.ds(tbase, TAIL)]
    trows = rows_v.at[0].at[pl.ds(0, TAIL)]
    pltpu.async_copy(feat_hbm.at[tidx], trows, gsems[0]).wait()
    for h in range(2):
        scatter_wait(N_BATCHES - 1, h)

    def tail_conv(t, _):
        r = t // (D // 32)
        g = (t % (D // 32)) * 32
        v = rows_v[0, r, pl.ds(g, 32)]
        a, b = plsc.unpack(v, format=plsc.PackFormat.INTERLEAVED)
        fbuf_v[0, r, pl.ds(g, 16)] = a
        fbuf_v[0, r, pl.ds(g + 16, 16)] = b
        return 0

    lax.fori_loop(0, TAIL * (D // 32), tail_conv, 0)
    tdidx = dst_v.at[pl.ds(tbase, TAIL)]
    pltpu.sync_copy(fbuf_v.at[0].at[pl.ds(0, TAIL)], acc_sh.at[tdidx],
                    add=True)
    plsc.subcore_barrier()

    ostage = fbuf_v.at[0].at[pl.ds(0, CHUNK)]
    for j in range(N_CHUNKS):
        r = row0 + j * CHUNK
        pltpu.sync_copy(acc_sh.at[pl.ds(r, CHUNK)], ostage)
        pltpu.sync_copy(ostage, out_hbm.at[c, pl.ds(r, CHUNK)])


def _linear_body(h0_ref, h1_ref, w_ref, b_ref, o_ref):
    h = h0_ref[0] + h1_ref[0]
    y = jnp.dot(h, w_ref[...], preferred_element_type=jnp.float32)
    o_ref[...] = jnp.maximum(y + b_ref[...], 0.0)


_ROW_BLK = 1000

_linear = pl.pallas_call(
    _linear_body,
    grid=(N_NODES // _ROW_BLK,),
    in_specs=[
        pl.BlockSpec((1, _ROW_BLK, D), lambda i: (0, i, 0)),
        pl.BlockSpec((1, _ROW_BLK, D), lambda i: (1, i, 0)),
        pl.BlockSpec((D, D), lambda i: (0, 0)),
        pl.BlockSpec((1, D), lambda i: (0, 0)),
    ],
    out_specs=pl.BlockSpec((_ROW_BLK, D), lambda i: (i, 0)),
    out_shape=jax.ShapeDtypeStruct((N_NODES, D), jnp.float32),
)


def kernel(features, edge_index, W, b):
    ei = edge_index.astype(jnp.int32)
    fb = features.astype(jnp.bfloat16)
    hp = _aggregate(ei, fb)
    return _linear(hp, hp, W[_PERM], b.reshape(1, D))


# speedup vs baseline: 1.7578x; 1.7578x over previous
import functools

import jax
import jax.numpy as jnp
from jax import lax
from jax.experimental import pallas as pl
from jax.experimental.pallas import tpu as pltpu
from jax.experimental.pallas import tpu_sc as plsc

N_NODES = 10000
N_EDGES = 320000
D = 128

NC = 2
NS = 16
N_TILES = NC * NS

EDGES_PER_TILE = N_EDGES // N_TILES
BATCH = 96
N_BATCHES = EDGES_PER_TILE // BATCH
TAIL = EDGES_PER_TILE - N_BATCHES * BATCH
N_PAD = 10240
ROWS_PER_TILE = N_PAD // NS
CHUNK = 80
N_CHUNKS = ROWS_PER_TILE // CHUNK


@functools.partial(
    pl.kernel,
    mesh=plsc.VectorSubcoreMesh(core_axis_name="c", subcore_axis_name="s"),
    compiler_params=pltpu.CompilerParams(use_tc_tiling_on_sc=False),
    out_type=jax.ShapeDtypeStruct((NC, N_PAD, D), jnp.float32),
    scratch_types=[
        pltpu.VMEM((EDGES_PER_TILE,), jnp.int32),
        pltpu.VMEM((EDGES_PER_TILE,), jnp.int32),
        pltpu.VMEM((2, BATCH, D), jnp.float32),
        pltpu.VMEM_SHARED((N_PAD, D), jnp.float32),
        [pltpu.SemaphoreType.DMA] * 2,
    ],
)
def _aggregate(ei_hbm, feat_hbm, out_hbm,
               src_v, dst_v, rows_v, acc_sh, gsems):
    c = lax.axis_index("c")
    s = lax.axis_index("s")
    w = c * NS + s

    zeros16 = jnp.zeros((16,), jnp.float32)

    def zero_body(i, _):
        r = i // (D // 16)
        col = (i % (D // 16)) * 16
        rows_v[0, r, pl.ds(col, 16)] = zeros16
        return 0

    lax.fori_loop(0, CHUNK * (D // 16), zero_body, 0)

    row0 = s * ROWS_PER_TILE
    zsrc = rows_v.at[0].at[pl.ds(0, CHUNK)]
    for j in range(N_CHUNKS):
        pltpu.sync_copy(zsrc, acc_sh.at[pl.ds(row0 + j * CHUNK, CHUNK)])

    ebase = w * EDGES_PER_TILE
    pltpu.sync_copy(ei_hbm.at[0, pl.ds(ebase, EDGES_PER_TILE)], src_v)
    pltpu.sync_copy(ei_hbm.at[1, pl.ds(ebase, EDGES_PER_TILE)], dst_v)
    plsc.subcore_barrier()

    def gather_start(i, slot):
        idx = src_v.at[pl.ds(i * BATCH, BATCH)]
        pltpu.async_copy(feat_hbm.at[idx], rows_v.at[slot], gsems[slot])

    def gather_wait(i, slot):
        idx = src_v.at[pl.ds(i * BATCH, BATCH)]
        pltpu.make_async_copy(feat_hbm.at[idx], rows_v.at[slot],
                              gsems[slot]).wait()

    gather_start(0, 0)
    gather_start(1, 1)

    def pair_body(k, _):
        for slot in range(2):
            i = 2 * k + slot
            gather_wait(i, slot)
            didx = dst_v.at[pl.ds(i * BATCH, BATCH)]
            pltpu.sync_copy(rows_v.at[slot], acc_sh.at[didx], add=True)

            @pl.when(i + 2 < N_BATCHES)
            def _():
                gather_start(i + 2, slot)
        return 0

    lax.fori_loop(0, N_BATCHES // 2, pair_body, 0)
    tidx = src_v.at[pl.ds(N_BATCHES * BATCH, TAIL)]
    trows = rows_v.at[0].at[pl.ds(0, TAIL)]
    pltpu.async_copy(feat_hbm.at[tidx], trows, gsems[0]).wait()
    tdidx = dst_v.at[pl.ds(N_BATCHES * BATCH, TAIL)]
    pltpu.sync_copy(trows, acc_sh.at[tdidx], add=True)
    plsc.subcore_barrier()

    pltpu.sync_copy(acc_sh.at[pl.ds(row0, ROWS_PER_TILE)],
                    out_hbm.at[c, pl.ds(row0, ROWS_PER_TILE)])


def _linear_body(h0_ref, h1_ref, w_ref, b_ref, o_ref):
    h = h0_ref[0] + h1_ref[0]
    y = jnp.dot(h, w_ref[...], preferred_element_type=jnp.float32)
    o_ref[...] = jnp.maximum(y + b_ref[...], 0.0)


_ROW_BLK = 1000

_linear = pl.pallas_call(
    _linear_body,
    grid=(N_NODES // _ROW_BLK,),
    in_specs=[
        pl.BlockSpec((1, _ROW_BLK, D), lambda i: (0, i, 0)),
        pl.BlockSpec((1, _ROW_BLK, D), lambda i: (1, i, 0)),
        pl.BlockSpec((D, D), lambda i: (0, 0)),
        pl.BlockSpec((1, D), lambda i: (0, 0)),
    ],
    out_specs=pl.BlockSpec((_ROW_BLK, D), lambda i: (i, 0)),
    out_shape=jax.ShapeDtypeStruct((N_NODES, D), jnp.float32),
)


def kernel(features, edge_index, W, b):
    ei = edge_index.astype(jnp.int32)
    hp = _aggregate(ei, features)
    return _linear(hp, hp, W, b.reshape(1, D))


# speedup vs baseline: 1.7919x; 1.0194x over previous
import functools

import jax
import jax.numpy as jnp
from jax import lax
from jax.experimental import pallas as pl
from jax.experimental.pallas import tpu as pltpu
from jax.experimental.pallas import tpu_sc as plsc

N_NODES = 10000
N_EDGES = 320000
D = 128

NC = 2
NS = 16
N_TILES = NC * NS

EDGES_PER_TILE = N_EDGES // N_TILES
BATCH = 96
N_BATCHES = EDGES_PER_TILE // BATCH
TAIL = EDGES_PER_TILE - N_BATCHES * BATCH
N_PAD = 10240
ROWS_PER_TILE = N_PAD // NS
CHUNK = 80
N_CHUNKS = ROWS_PER_TILE // CHUNK


@functools.partial(
    pl.kernel,
    mesh=plsc.VectorSubcoreMesh(core_axis_name="c", subcore_axis_name="s"),
    compiler_params=pltpu.CompilerParams(use_tc_tiling_on_sc=False),
    out_type=jax.ShapeDtypeStruct((NC, N_PAD, D), jnp.float32),
    scratch_types=[
        pltpu.VMEM((EDGES_PER_TILE,), jnp.int32),
        pltpu.VMEM((EDGES_PER_TILE,), jnp.int32),
        pltpu.VMEM((2, BATCH, D), jnp.float32),
        pltpu.VMEM_SHARED((N_PAD, D), jnp.float32),
        [pltpu.SemaphoreType.DMA] * 2,
        pltpu.SemaphoreType.DMA,
    ],
)
def _aggregate(ei_hbm, feat_hbm, out_hbm,
               src_v, dst_v, rows_v, acc_sh, gsems, psem):
    c = lax.axis_index("c")
    s = lax.axis_index("s")
    w = c * NS + s

    ebase = w * EDGES_PER_TILE
    pltpu.async_copy(ei_hbm.at[0, pl.ds(ebase, EDGES_PER_TILE)], src_v, psem)
    pltpu.async_copy(ei_hbm.at[1, pl.ds(ebase, EDGES_PER_TILE)], dst_v, psem)

    zeros16 = jnp.zeros((16,), jnp.float32)

    def zero_body(i, _):
        r = i // (D // 16)
        col = (i % (D // 16)) * 16
        rows_v[0, r, pl.ds(col, 16)] = zeros16
        return 0

    lax.fori_loop(0, CHUNK * (D // 16), zero_body, 0)

    row0 = s * ROWS_PER_TILE
    zsrc = rows_v.at[0].at[pl.ds(0, CHUNK)]
    for j in range(N_CHUNKS):
        pltpu.async_copy(zsrc, acc_sh.at[pl.ds(row0 + j * CHUNK, CHUNK)], psem)
    for j in range(N_CHUNKS):
        pltpu.make_async_copy(zsrc, acc_sh.at[pl.ds(row0 + j * CHUNK, CHUNK)],
                              psem).wait()
    pltpu.make_async_copy(ei_hbm.at[0, pl.ds(ebase, EDGES_PER_TILE)], src_v,
                          psem).wait()
    pltpu.make_async_copy(ei_hbm.at[1, pl.ds(ebase, EDGES_PER_TILE)], dst_v,
                          psem).wait()
    plsc.subcore_barrier()

    def gather_start(i, slot):
        idx = src_v.at[pl.ds(i * BATCH, BATCH)]
        pltpu.async_copy(feat_hbm.at[idx], rows_v.at[slot], gsems[slot])

    def gather_wait(i, slot):
        idx = src_v.at[pl.ds(i * BATCH, BATCH)]
        pltpu.make_async_copy(feat_hbm.at[idx], rows_v.at[slot],
                              gsems[slot]).wait()

    gather_start(0, 0)
    gather_start(1, 1)

    def pair_body(k, _):
        for slot in range(2):
            i = 2 * k + slot
            gather_wait(i, slot)
            didx = dst_v.at[pl.ds(i * BATCH, BATCH)]
            pltpu.sync_copy(rows_v.at[slot], acc_sh.at[didx], add=True)

            @pl.when(i + 2 < N_BATCHES)
            def _():
                gather_start(i + 2, slot)
        return 0

    lax.fori_loop(0, N_BATCHES // 2, pair_body, 0)
    tidx = src_v.at[pl.ds(N_BATCHES * BATCH, TAIL)]
    trows = rows_v.at[0].at[pl.ds(0, TAIL)]
    pltpu.async_copy(feat_hbm.at[tidx], trows, gsems[0]).wait()
    tdidx = dst_v.at[pl.ds(N_BATCHES * BATCH, TAIL)]
    pltpu.sync_copy(trows, acc_sh.at[tdidx], add=True)
    plsc.subcore_barrier()

    pltpu.sync_copy(acc_sh.at[pl.ds(row0, ROWS_PER_TILE)],
                    out_hbm.at[c, pl.ds(row0, ROWS_PER_TILE)])


def _linear_body(h0_ref, h1_ref, w_ref, b_ref, o_ref):
    h = h0_ref[0] + h1_ref[0]
    y = jnp.dot(h, w_ref[...], preferred_element_type=jnp.float32)
    o_ref[...] = jnp.maximum(y + b_ref[...], 0.0)


_ROW_BLK = 1000

_linear = pl.pallas_call(
    _linear_body,
    grid=(N_NODES // _ROW_BLK,),
    in_specs=[
        pl.BlockSpec((1, _ROW_BLK, D), lambda i: (0, i, 0)),
        pl.BlockSpec((1, _ROW_BLK, D), lambda i: (1, i, 0)),
        pl.BlockSpec((D, D), lambda i: (0, 0)),
        pl.BlockSpec((1, D), lambda i: (0, 0)),
    ],
    out_specs=pl.BlockSpec((_ROW_BLK, D), lambda i: (i, 0)),
    out_shape=jax.ShapeDtypeStruct((N_NODES, D), jnp.float32),
)


def kernel(features, edge_index, W, b):
    ei = edge_index.astype(jnp.int32)
    hp = _aggregate(ei, features)
    return _linear(hp, hp, W, b.reshape(1, D))
